# baseline (speedup 1.0000x reference)
import functools

import jax
import jax.numpy as jnp
from jax import lax
from jax.experimental import pallas as pl
from jax.experimental.pallas import tpu as pltpu
from jax.experimental.pallas import tpu_sc as plsc

_SMOOTH = 0.1
_B_SC = 4096
_RT = 16


def _chunks(C):
    full, rem = divmod(C, 128)
    out = [(k * 128, None) for k in range(full)]
    if rem:
        out.append((C - 128, full * 128))
    return out


def _loss_kernel(x_ref, t_ref, ps_ref):
    R, C = x_ref.shape
    t = t_ref[0, 0, :]
    tcol = t[:, None]

    m = jnp.full((R, 128), -3.0e38, dtype=jnp.float32)
    sx = jnp.zeros((R, 128), dtype=jnp.float32)
    xt = jnp.zeros((R, 128), dtype=jnp.float32)
    for off, mask_from in _chunks(C):
        xc = x_ref[:, off:off + 128]
        cols = lax.broadcasted_iota(jnp.int32, (R, 128), 1) + off
        hit = cols == tcol
        if mask_from is not None:
            valid = cols >= mask_from
            xc = jnp.where(valid, xc, 0.0)
            m = jnp.maximum(m, jnp.where(valid, xc, -3.0e38))
            hit = hit & valid
        else:
            m = jnp.maximum(m, xc)
        sx = sx + xc
        xt = xt + jnp.where(hit, xc, 0.0)
    mrow = jnp.max(m, axis=1, keepdims=True)
    s_row = jnp.sum(sx, axis=1)
    xt_row = jnp.sum(xt, axis=1)

    e = jnp.zeros((R, 128), dtype=jnp.float32)
    for off, mask_from in _chunks(C):
        xc = x_ref[:, off:off + 128]
        if mask_from is not None:
            cols = lax.broadcasted_iota(jnp.int32, (R, 128), 1) + off
            xc = jnp.where(cols >= mask_from, xc, -3.0e38)
        e = e + jnp.exp(xc - mrow)
    lse = jnp.log(jnp.sum(e, axis=1)) + mrow[:, 0]

    a = 1.0 - _SMOOTH - _SMOOTH / (C - 1)
    b = _SMOOTH / (C - 1)
    ps_ref[0, 0, :] = lse - a * xt_row - b * s_row


def _sc_loss(B, C, row0):
    info = plsc.get_sparse_core_info()
    NC, NS = info.num_cores, info.num_subcores
    NW = NC * NS
    rpw = _B_SC // NW
    ntiles = rpw // _RT
    a = 1.0 - _SMOOTH - _SMOOTH / (C - 1)
    b = _SMOOTH / (C - 1)
    nfull = C // 16
    rem = C % 16
    tail_off = C - 16
    amax = ((C - 16) // 16) * 16

    mesh = plsc.VectorSubcoreMesh(core_axis_name="c", subcore_axis_name="s")

    @functools.partial(
        pl.kernel, mesh=mesh,
        compiler_params=pltpu.CompilerParams(use_tc_tiling_on_sc=True),
        out_type=[jax.ShapeDtypeStruct((_B_SC * 16,), jnp.float32),
                  jax.ShapeDtypeStruct((_B_SC * 16,), jnp.float32),
                  jax.ShapeDtypeStruct((_B_SC * 16,), jnp.float32)],
        scratch_types=[pltpu.VMEM((_RT, C), jnp.float32),
                       pltpu.VMEM((_RT,), jnp.int32),
                       pltpu.VMEM((rpw * 16,), jnp.float32),
                       pltpu.VMEM((rpw * 16,), jnp.float32),
                       pltpu.VMEM((rpw * 16,), jnp.float32)],
    )
    def sc_k(x_hbm, t_hbm, m_hbm, s_hbm, w_hbm, xbuf, tbuf, mbuf, sbuf, wbuf):
        wid = lax.axis_index("s") * NC + lax.axis_index("c")
        base = wid * rpw
        lanes = lax.broadcasted_iota(jnp.int32, (16,), 0)
        tail_mask = lanes >= (16 - rem)
        neg = jnp.float32(-3.0e38)

        def tile_body(tile, _):
            r0 = row0 + base + tile * _RT
            pltpu.sync_copy(x_hbm.at[pl.ds(r0, _RT), :], xbuf)
            pltpu.sync_copy(t_hbm.at[pl.ds(r0, _RT)], tbuf)
            tv = tbuf[...]

            for k in range(_RT):
                def c1(c, m16):
                    return jnp.maximum(m16, xbuf[k, pl.ds(c * 16, 16)])

                m16 = lax.fori_loop(0, nfull, c1, jnp.full((16,), neg))
                xtail = xbuf[k, pl.ds(tail_off, 16)]
                m16 = jnp.maximum(m16, jnp.where(tail_mask, xtail, neg))

                def c2(c, sc):
                    s16, sx16 = sc
                    xc = xbuf[k, pl.ds(c * 16, 16)]
                    return s16 + jnp.exp(xc - m16), sx16 + xc

                s16, sx16 = lax.fori_loop(
                    0, nfull, c2,
                    (jnp.zeros((16,), jnp.float32),
                     jnp.zeros((16,), jnp.float32)))
                s16 = s16 + jnp.where(tail_mask, jnp.exp(xtail - m16), 0.0)
                sx16 = sx16 + jnp.where(tail_mask, xtail, 0.0)

                tk = tv[k]
                off_k = jnp.minimum((tk // 16) * 16, amax)
                xa = xbuf[k, pl.ds(off_k, 16)]
                in_tail = tk >= amax + 16
                xsel = jnp.where(in_tail, xtail, xa)
                base_col = jnp.where(in_tail, tail_off, off_k)
                xtv = jnp.where(lanes == tk - base_col, xsel, 0.0)

                row = tile * _RT + k
                mbuf[pl.ds(row * 16, 16)] = m16
                sbuf[pl.ds(row * 16, 16)] = s16
                wbuf[pl.ds(row * 16, 16)] = -a * xtv - b * sx16
            return 0

        lax.fori_loop(0, ntiles, tile_body, 0)
        pltpu.sync_copy(mbuf, m_hbm.at[pl.ds(base * 16, rpw * 16)])
        pltpu.sync_copy(sbuf, s_hbm.at[pl.ds(base * 16, rpw * 16)])
        pltpu.sync_copy(wbuf, w_hbm.at[pl.ds(base * 16, rpw * 16)])

    return sc_k


def _select_kernel(ps_ref, m_ref, s_ref, w_ref, o_ref, *, keep):
    v_tc = ps_ref[:, 0, :]
    m2 = m_ref[...]
    M = jnp.max(m2, axis=1)
    s_scaled = jnp.sum(s_ref[...] * jnp.exp(m2 - M[:, None]), axis=1)
    v_sc = jnp.log(s_scaled) + M + jnp.sum(w_ref[...], axis=1)

    def skey(v):
        bits = lax.bitcast_convert_type(v, jnp.int32)
        return jnp.where(bits < 0, bits ^ jnp.int32(0x7FFFFFFF), bits)

    k_tc, k_sc = skey(v_tc), skey(v_sc)

    def body(j, prefix):
        cand = prefix + (jnp.int32(1) << jnp.int32(31 - j))
        cnt = (jnp.sum((k_tc >= cand).astype(jnp.int32))
               + jnp.sum((k_sc >= cand).astype(jnp.int32)))
        return jnp.where(cnt >= keep, cand, prefix)

    kth = lax.fori_loop(0, 32, body, jnp.int32(-2147483648))
    tau_bits = jnp.where(kth < 0, kth ^ jnp.int32(0x7FFFFFFF), kth)
    tau = lax.bitcast_convert_type(tau_bits, jnp.float32)
    cnt_gt = (jnp.sum((k_tc > kth).astype(jnp.int32))
              + jnp.sum((k_sc > kth).astype(jnp.int32)))
    sum_gt = (jnp.sum(jnp.where(k_tc > kth, v_tc, 0.0))
              + jnp.sum(jnp.where(k_sc > kth, v_sc, 0.0)))
    total = sum_gt + (keep - cnt_gt).astype(jnp.float32) * tau
    o_ref[...] = jnp.reshape(total / keep, (1, 1))


def kernel(input, target):
    B, C = input.shape
    R = 1024
    b_tc = B - _B_SC
    G = b_tc // R
    keep = min(B, int(B * 0.7))
    t32 = target.astype(jnp.int32)
    t3 = t32[:b_tc].reshape(G, 1, R)

    ps_tc = pl.pallas_call(
        _loss_kernel,
        grid=(G,),
        in_specs=[
            pl.BlockSpec((R, C), lambda i: (i, 0)),
            pl.BlockSpec((1, 1, R), lambda i: (i, 0, 0)),
        ],
        out_specs=pl.BlockSpec((1, 1, R), lambda i: (i, 0, 0)),
        out_shape=jax.ShapeDtypeStruct((G, 1, R), jnp.float32),
        compiler_params=pltpu.CompilerParams(
            dimension_semantics=("parallel",)),
    )(input, t3)

    m_sc, s_sc, w_sc = _sc_loss(B, C, b_tc)(input, t32)

    out = pl.pallas_call(
        functools.partial(_select_kernel, keep=keep),
        out_shape=jax.ShapeDtypeStruct((1, 1), jnp.float32),
    )(ps_tc, m_sc.reshape(_B_SC, 16), s_sc.reshape(_B_SC, 16),
      w_sc.reshape(_B_SC, 16))
    return out[0, 0]


# speedup vs baseline: 1.7157x; 1.7157x over previous
import functools

import jax
import jax.numpy as jnp
from jax import lax
from jax.experimental import pallas as pl
from jax.experimental.pallas import tpu as pltpu

_SMOOTH = 0.1
_NBUF = 6


def _chunks(C):
    full, rem = divmod(C, 128)
    out = [(k * 128, None) for k in range(full)]
    if rem:
        out.append((C - 128, full * 128))
    return out


def _ohem_kernel(x_hbm, t_ref, o_ref, xbuf, ps_ref, sem, *, nblocks, keep):
    i = pl.program_id(0)
    NB, R, C = xbuf.shape

    def start(blk):
        slot = lax.rem(blk, NB)
        pltpu.make_async_copy(
            x_hbm.at[pl.ds(blk * R, R), :], xbuf.at[slot], sem.at[slot]
        ).start()

    @pl.when(i == 0)
    def _prologue():
        for j in range(_NBUF):
            start(jnp.int32(j))

    slot = lax.rem(i, NB)
    pltpu.make_async_copy(
        x_hbm.at[pl.ds(i * R, R), :], xbuf.at[slot], sem.at[slot]
    ).wait()

    t = t_ref[0, 0, :]
    tcol = t[:, None]
    m = jnp.full((R, 128), -3.0e38, dtype=jnp.float32)
    sx = jnp.zeros((R, 128), dtype=jnp.float32)
    xt = jnp.zeros((R, 128), dtype=jnp.float32)
    for off, mask_from in _chunks(C):
        xc = xbuf[slot, :, off:off + 128]
        cols = lax.broadcasted_iota(jnp.int32, (R, 128), 1) + off
        hit = cols == tcol
        if mask_from is not None:
            valid = cols >= mask_from
            xc = jnp.where(valid, xc, 0.0)
            m = jnp.maximum(m, jnp.where(valid, xc, -3.0e38))
            hit = hit & valid
        else:
            m = jnp.maximum(m, xc)
        sx = sx + xc
        xt = xt + jnp.where(hit, xc, 0.0)
    mrow = jnp.max(m, axis=1, keepdims=True)
    s_row = jnp.sum(sx, axis=1)
    xt_row = jnp.sum(xt, axis=1)

    e = jnp.zeros((R, 128), dtype=jnp.float32)
    for off, mask_from in _chunks(C):
        xc = xbuf[slot, :, off:off + 128]
        if mask_from is not None:
            cols = lax.broadcasted_iota(jnp.int32, (R, 128), 1) + off
            xc = jnp.where(cols >= mask_from, xc, -3.0e38)
        e = e + jnp.exp(xc - mrow)
    lse = jnp.log(jnp.sum(e, axis=1)) + mrow[:, 0]

    a = 1.0 - _SMOOTH - _SMOOTH / (C - 1)
    b = _SMOOTH / (C - 1)
    ps_ref[i, :] = lse - a * xt_row - b * s_row

    @pl.when(i + _NBUF < nblocks)
    def _refill():
        start(i + _NBUF)

    @pl.when(i == nblocks - 1)
    def _select():
        v = ps_ref[...]
        bits = lax.bitcast_convert_type(v, jnp.int32)
        skey = jnp.where(bits < 0, bits ^ jnp.int32(0x7FFFFFFF), bits)

        def body(j, prefix):
            cand = prefix + (jnp.int32(1) << jnp.int32(31 - j))
            cnt = jnp.sum((skey >= cand).astype(jnp.int32))
            return jnp.where(cnt >= keep, cand, prefix)

        kth = lax.fori_loop(0, 32, body, jnp.int32(-2147483648))
        tau_bits = jnp.where(kth < 0, kth ^ jnp.int32(0x7FFFFFFF), kth)
        tau = lax.bitcast_convert_type(tau_bits, jnp.float32)
        gt = skey > kth
        cnt_gt = jnp.sum(gt.astype(jnp.int32))
        sum_gt = jnp.sum(jnp.where(gt, v, 0.0))
        total = sum_gt + (keep - cnt_gt).astype(jnp.float32) * tau
        o_ref[...] = jnp.reshape(total / keep, (1, 1))


def kernel(input, target):
    B, C = input.shape
    R = 512
    G = B // R
    keep = min(B, int(B * 0.7))
    t3 = target.astype(jnp.int32).reshape(G, 1, R)
    out = pl.pallas_call(
        functools.partial(_ohem_kernel, nblocks=G, keep=keep),
        grid=(G,),
        in_specs=[
            pl.BlockSpec(memory_space=pl.ANY),
            pl.BlockSpec((1, 1, R), lambda i: (i, 0, 0)),
        ],
        out_specs=pl.BlockSpec((1, 1), lambda i: (0, 0)),
        out_shape=jax.ShapeDtypeStruct((1, 1), jnp.float32),
        scratch_shapes=[
            pltpu.VMEM((_NBUF, R, C), jnp.float32),
            pltpu.VMEM((G, R), jnp.float32),
            pltpu.SemaphoreType.DMA((_NBUF,)),
        ],
    )(input, t3)
    return out[0, 0]


# speedup vs baseline: 1.7773x; 1.0359x over previous
import functools

import jax
import jax.numpy as jnp
from jax import lax
from jax.experimental import pallas as pl
from jax.experimental.pallas import tpu as pltpu

_SMOOTH = 0.1
_NBUF = 4


def _chunks(C):
    full, rem = divmod(C, 128)
    out = [(k * 128, None) for k in range(full)]
    if rem:
        out.append((C - 128, full * 128))
    return out


def _ohem_kernel(x_hbm, t_ref, o_ref, xbuf, ps_ref, sem, *, nblocks, keep):
    i = pl.program_id(0)
    NB, R, C = xbuf.shape

    def start(blk):
        slot = lax.rem(blk, NB)
        pltpu.make_async_copy(
            x_hbm.at[pl.ds(blk * R, R), :], xbuf.at[slot], sem.at[slot]
        ).start()

    @pl.when(i == 0)
    def _prologue():
        for j in range(_NBUF):
            start(jnp.int32(j))

    slot = lax.rem(i, NB)
    pltpu.make_async_copy(
        x_hbm.at[pl.ds(i * R, R), :], xbuf.at[slot], sem.at[slot]
    ).wait()

    t = t_ref[0, 0, :]
    tcol = t[:, None]
    m = jnp.full((R, 128), -3.0e38, dtype=jnp.float32)
    sx = jnp.zeros((R, 128), dtype=jnp.float32)
    xt = jnp.zeros((R, 128), dtype=jnp.float32)
    for off, mask_from in _chunks(C):
        xc = xbuf[slot, :, off:off + 128]
        cols = lax.broadcasted_iota(jnp.int32, (R, 128), 1) + off
        hit = cols == tcol
        if mask_from is not None:
            valid = cols >= mask_from
            xc = jnp.where(valid, xc, 0.0)
            m = jnp.maximum(m, jnp.where(valid, xc, -3.0e38))
            hit = hit & valid
        else:
            m = jnp.maximum(m, xc)
        sx = sx + xc
        xt = xt + jnp.where(hit, xc, 0.0)
    mrow = jnp.max(m, axis=1, keepdims=True)
    s_row = jnp.sum(sx, axis=1)
    xt_row = jnp.sum(xt, axis=1)

    e = jnp.zeros((R, 128), dtype=jnp.float32)
    for off, mask_from in _chunks(C):
        xc = xbuf[slot, :, off:off + 128]
        if mask_from is not None:
            cols = lax.broadcasted_iota(jnp.int32, (R, 128), 1) + off
            xc = jnp.where(cols >= mask_from, xc, -3.0e38)
        e = e + jnp.exp(xc - mrow)
    lse = jnp.log(jnp.sum(e, axis=1)) + mrow[:, 0]

    a = 1.0 - _SMOOTH - _SMOOTH / (C - 1)
    b = _SMOOTH / (C - 1)
    ps_ref[i, :] = lse - a * xt_row - b * s_row

    @pl.when(i + _NBUF < nblocks)
    def _refill():
        start(i + _NBUF)

    @pl.when(i == nblocks - 1)
    def _select():
        v = ps_ref[...]
        bits = lax.bitcast_convert_type(v, jnp.int32)
        skey = jnp.where(bits < 0, bits ^ jnp.int32(0x7FFFFFFF), bits)

        def body(j, prefix):
            cand = prefix + (jnp.int32(1) << jnp.int32(31 - j))
            cnt = jnp.sum((skey >= cand).astype(jnp.int32))
            return jnp.where(cnt >= keep, cand, prefix)

        kth = lax.fori_loop(0, 32, body, jnp.int32(-2147483648))
        tau_bits = jnp.where(kth < 0, kth ^ jnp.int32(0x7FFFFFFF), kth)
        tau = lax.bitcast_convert_type(tau_bits, jnp.float32)
        gt = skey > kth
        cnt_gt = jnp.sum(gt.astype(jnp.int32))
        sum_gt = jnp.sum(jnp.where(gt, v, 0.0))
        total = sum_gt + (keep - cnt_gt).astype(jnp.float32) * tau
        o_ref[...] = jnp.reshape(total / keep, (1, 1))


def kernel(input, target):
    B, C = input.shape
    R = 1024
    G = B // R
    keep = min(B, int(B * 0.7))
    t3 = target.astype(jnp.int32).reshape(G, 1, R)
    out = pl.pallas_call(
        functools.partial(_ohem_kernel, nblocks=G, keep=keep),
        grid=(G,),
        in_specs=[
            pl.BlockSpec(memory_space=pl.ANY),
            pl.BlockSpec((1, 1, R), lambda i: (i, 0, 0)),
        ],
        out_specs=pl.BlockSpec((1, 1), lambda i: (0, 0)),
        out_shape=jax.ShapeDtypeStruct((1, 1), jnp.float32),
        scratch_shapes=[
            pltpu.VMEM((_NBUF, R, C), jnp.float32),
            pltpu.VMEM((G, R), jnp.float32),
            pltpu.SemaphoreType.DMA((_NBUF,)),
        ],
    )(input, t3)
    return out[0, 0]
